# baseline (speedup 1.0000x reference)
import functools

import jax
import jax.numpy as jnp
from jax import lax
from jax.experimental import pallas as pl
from jax.experimental.pallas import tpu as pltpu
from jax.experimental.pallas import tpu_sc as plsc

_NUM_CORES = 2
_NUM_SUBCORES = 16
_NUM_WORKERS = _NUM_CORES * _NUM_SUBCORES


def _copy_body(k_in, v_in, k_out, v_out, sem_k, sem_v, *, rows_per_worker):
    wid = lax.axis_index("s") * _NUM_CORES + lax.axis_index("c")
    base = wid * rows_per_worker
    sl = pl.ds(base, rows_per_worker)
    ck = pltpu.async_copy(k_in.at[sl], k_out.at[sl], sem_k)
    cv = pltpu.async_copy(v_in.at[sl], v_out.at[sl], sem_v)
    ck.wait()
    cv.wait()


def kernel(k, v, k_cache, v_cache):
    L = k.shape[0]
    rows_per_worker = L // _NUM_WORKERS
    assert rows_per_worker * _NUM_WORKERS == L
    mesh = plsc.VectorSubcoreMesh(core_axis_name="c", subcore_axis_name="s")
    body = functools.partial(_copy_body, rows_per_worker=rows_per_worker)
    k_out, v_out = pl.kernel(
        body,
        out_type=[
            jax.ShapeDtypeStruct(k.shape, k.dtype),
            jax.ShapeDtypeStruct(v.shape, v.dtype),
        ],
        mesh=mesh,
        scratch_types=[pltpu.SemaphoreType.DMA, pltpu.SemaphoreType.DMA],
    )(k, v)
    return (k_out, v_out)


# speedup vs baseline: 31.3159x; 31.3159x over previous
import functools

import jax
import jax.numpy as jnp
from jax import lax
from jax.experimental import pallas as pl
from jax.experimental.pallas import tpu as pltpu
from jax.experimental.pallas import tpu_sc as plsc

_NUM_CORES = 2
_NUM_SUBCORES = 16
_NUM_WORKERS = _NUM_CORES * _NUM_SUBCORES
_NBUF = 3
_CHUNK_ROWS = 16


def _copy_body(k_in, v_in, k_out, v_out, *scratch, rows_per_worker):
    bufs = scratch[:_NBUF]
    gsems = scratch[_NBUF:2 * _NBUF]
    ssems = scratch[2 * _NBUF:3 * _NBUF]
    wid = lax.axis_index("s") * _NUM_CORES + lax.axis_index("c")
    base = wid * rows_per_worker

    n_per_tensor = rows_per_worker // _CHUNK_ROWS
    chunks = []
    for src, dst in ((k_in, k_out), (v_in, v_out)):
        for c in range(n_per_tensor):
            chunks.append((src, dst, c * _CHUNK_ROWS))
    n = len(chunks)

    gathers = [None] * _NBUF
    for j in range(min(_NBUF, n)):
        src, _, off = chunks[j]
        sl = pl.ds(base + off, _CHUNK_ROWS)
        gathers[j] = pltpu.async_copy(src.at[sl], bufs[j], gsems[j])

    for j in range(n):
        b = j % _NBUF
        _, dst, off = chunks[j]
        sl = pl.ds(base + off, _CHUNK_ROWS)
        gathers[b].wait()
        scat = pltpu.async_copy(bufs[b], dst.at[sl], ssems[b])
        jn = j + _NBUF
        if jn < n:
            src_n, _, off_n = chunks[jn]
            sl_n = pl.ds(base + off_n, _CHUNK_ROWS)
            scat.wait()
            gathers[b] = pltpu.async_copy(src_n.at[sl_n], bufs[b], gsems[b])
        else:
            scat.wait()


def kernel(k, v, k_cache, v_cache):
    L, H, D = k.shape
    rows_per_worker = L // _NUM_WORKERS
    assert rows_per_worker * _NUM_WORKERS == L
    assert rows_per_worker % _CHUNK_ROWS == 0
    mesh = plsc.VectorSubcoreMesh(core_axis_name="c", subcore_axis_name="s")
    body = functools.partial(_copy_body, rows_per_worker=rows_per_worker)
    scratch = (
        [pltpu.VMEM((_CHUNK_ROWS, H, D), k.dtype) for _ in range(_NBUF)]
        + [pltpu.SemaphoreType.DMA for _ in range(2 * _NBUF)]
    )
    k_out, v_out = pl.kernel(
        body,
        out_type=[
            jax.ShapeDtypeStruct(k.shape, k.dtype),
            jax.ShapeDtypeStruct(v.shape, v.dtype),
        ],
        mesh=mesh,
        scratch_types=scratch,
    )(k, v)
    return (k_out, v_out)
